# baseline (speedup 1.0000x reference)
import jax
import jax.numpy as jnp
from jax.experimental import pallas as pl
from jax.experimental.pallas import tpu as pltpu

B = 32
P = 784
OUTW = 1568
ROWS = (B * OUTW) // 128


def _body(rf_ref, idx_ref, out_ref):
    rf = rf_ref[:, :]
    sflat = idx_ref[:, :]
    rr = jax.lax.broadcasted_iota(jnp.int32, (ROWS, 128), 0)
    ll = jax.lax.broadcasted_iota(jnp.int32, (ROWS, 128), 1)
    g = rr * 128 + ll
    b = g // OUTW
    jpar = ll & 1
    tbl = jnp.broadcast_to(sflat, (ROWS, 128))
    sel = jnp.take_along_axis(tbl, 2 * b + jpar, axis=1)
    gidx = (ll & ~1) + sel
    out_ref[:, :] = jnp.take_along_axis(rf, gidx, axis=1)


@jax.jit
def kernel(reals, fakes, shuffle_indices):
    reals2 = reals.astype(jnp.float32).reshape(B, P)
    fakes2 = fakes.astype(jnp.float32).reshape(B, P)
    rf = jnp.stack([fakes2, reals2], axis=-1).reshape(ROWS, 128)
    idx = shuffle_indices.astype(jnp.int32)
    idxp = jnp.zeros((1, 128), jnp.int32).at[0, :64].set(idx.reshape(64))
    out = pl.pallas_call(
        _body,
        out_shape=jax.ShapeDtypeStruct((ROWS, 128), jnp.float32),
    )(rf, idxp)
    return out.reshape(B, 28, 28, 2, 1)


# speedup vs baseline: 1.6022x; 1.6022x over previous
import jax
import jax.numpy as jnp
from jax.experimental import pallas as pl
from jax.experimental.pallas import tpu as pltpu

B = 32
P = 784
OUTW = 1568
SR = (B * P) // 128


def _body(f_ref, r_ref, s_ref, out_ref):
    f = f_ref[:, :]
    r = r_ref[:, :]
    tblb = jnp.broadcast_to(s_ref[:, :], (SR, 128))
    ll = jax.lax.broadcasted_iota(jnp.int32, (SR, 128), 1)
    rr = jax.lax.broadcasted_iota(jnp.int32, (SR, 128), 0)
    half = ll >> 1
    jpar = ll & 1
    for h in (0, 1):
        src = half + 64 * h
        vf = jnp.take_along_axis(f, src, axis=1)
        vr = jnp.take_along_axis(r, src, axis=1)
        g = rr * 256 + 128 * h + ll
        sel = jnp.take_along_axis(tblb, 2 * (g // OUTW) + jpar, axis=1)
        out_ref[:, 128 * h:128 * (h + 1)] = jnp.where(sel == 1, vr, vf)


@jax.jit
def kernel(reals, fakes, shuffle_indices):
    reals2 = reals.astype(jnp.float32).reshape(SR, 128)
    fakes2 = fakes.astype(jnp.float32).reshape(SR, 128)
    idx = shuffle_indices.astype(jnp.int32)
    idxp = jnp.zeros((1, 128), jnp.int32).at[0, :64].set(idx.reshape(64))
    out = pl.pallas_call(
        _body,
        out_shape=jax.ShapeDtypeStruct((SR, 256), jnp.float32),
    )(fakes2, reals2, idxp)
    return out.reshape(B, 28, 28, 2, 1)


# speedup vs baseline: 4.0393x; 2.5211x over previous
import jax
import jax.numpy as jnp
from jax.experimental import pallas as pl
from jax.experimental.pallas import tpu as pltpu

B = 32
H = 28
W = 28


def _body(f_ref, r_ref, s_ref, o_ref):
    for b in range(B):
        fb = f_ref[b]
        rb = r_ref[b]
        for j in range(2):
            o_ref[b, j] = jnp.where(s_ref[b, j] == 1, rb, fb)


@jax.jit
def kernel(reals, fakes, shuffle_indices):
    f3 = fakes.reshape(B, H, W)
    r3 = reals.reshape(B, H, W)
    out = pl.pallas_call(
        _body,
        out_shape=jax.ShapeDtypeStruct((B, 2, H, W), jnp.float32),
        in_specs=[
            pl.BlockSpec(memory_space=pltpu.MemorySpace.VMEM),
            pl.BlockSpec(memory_space=pltpu.MemorySpace.VMEM),
            pl.BlockSpec(memory_space=pltpu.MemorySpace.SMEM),
        ],
    )(f3, r3, shuffle_indices)
    return out.transpose(0, 2, 3, 1)[:, :, :, :, None]


# speedup vs baseline: 15.9500x; 3.9487x over previous
import jax
import jax.numpy as jnp
from jax.experimental import pallas as pl
from jax.experimental.pallas import tpu as pltpu

B = 32
P = 784


def _body(f_ref, r_ref, s_ref, o_ref):
    f = f_ref[:, :]
    r = r_ref[:, :]
    m0 = s_ref[0:1, :] == 1
    m1 = s_ref[1:2, :] == 1
    o0 = jnp.where(m0, r, f)
    o1 = jnp.where(m1, r, f)
    o_ref[:, :] = jnp.stack([o0, o1], axis=1).reshape(2 * P, B)


@jax.jit
def kernel(reals, fakes, shuffle_indices):
    f2 = fakes.reshape(B, P).transpose(1, 0)
    r2 = reals.reshape(B, P).transpose(1, 0)
    s2 = shuffle_indices.transpose(1, 0)
    out = pl.pallas_call(
        _body,
        out_shape=jax.ShapeDtypeStruct((2 * P, B), jnp.float32),
    )(f2, r2, s2)
    return out.reshape(28, 28, 2, B).transpose(3, 0, 1, 2)[:, :, :, :, None]
